# baseline (speedup 1.0000x reference)
import jax
import jax.numpy as jnp
from jax import lax
from jax.experimental import pallas as pl


_B, _C, _H, _W = 8, 1024, 24, 24
_HW = _H * _W
_R = _C // 2
_L = 2 * _HW


def _argmax_body(x_ref, out_ref):
    x = x_ref[0]
    m = jnp.max(x, axis=0, keepdims=True)
    iota = lax.broadcasted_iota(jnp.int32, (_R, _L), 0)
    r = jnp.min(jnp.where(x == m, iota, _R), axis=0, keepdims=True)
    v0, v1 = m[:, :_HW], m[:, _HW:]
    c0 = 2 * r[:, :_HW]
    c1 = 2 * r[:, _HW:] + 1
    take1 = (v1 > v0) | ((v1 == v0) & (c1 < c0))
    out_ref[0] = jnp.where(take1, c1, c0)


def kernel(inputs):
    x3 = inputs.reshape(_B, _R, _L)
    idx = pl.pallas_call(
        _argmax_body,
        grid=(_B,),
        in_specs=[pl.BlockSpec((1, _R, _L), lambda b: (b, 0, 0))],
        out_specs=pl.BlockSpec((1, 1, _HW), lambda b: (b, 0, 0)),
        out_shape=jax.ShapeDtypeStruct((_B, 1, _HW), jnp.int32),
    )(x3)
    return (inputs, idx.reshape(_B, 1, _H, _W))
